# Initial kernel scaffold; baseline (speedup 1.0000x reference)
import jax
import jax.numpy as jnp
from jax.experimental import pallas as pl


def kernel(positions, velocities, radii, masses):
    raise NotImplementedError("write your pallas kernel here")


# speedup vs baseline: 418.2552x; 418.2552x over previous
import functools

import jax
import jax.numpy as jnp
from jax import lax
from jax.experimental import pallas as pl
from jax.experimental.pallas import tpu as pltpu
from jax.experimental.pallas import tpu_sc as plsc

_N = 1024
_B = 256
_L = 16


def _shift_right(x, s, zero):
    pad = jnp.full((x.shape[0], s), zero, x.dtype)
    return jnp.concatenate([pad, x[:, :-s]], axis=1)


def _detect_body(omu_ref, pxr, pyr, rr, mr, pxc, pyc, rc, mc,
                 js_ref, nx_ref, ny_ref, anx_ref, any_ref, bnx_ref, bny_ref,
                 dpix_ref, dpiy_ref, dpjx_ref, dpjy_ref):
    omu = omu_ref[0, 0]
    i0 = pl.program_id(0) * _B
    jj = lax.broadcasted_iota(jnp.int32, (_B, _N), 1)
    ii = i0 + lax.broadcasted_iota(jnp.int32, (_B, _N), 0)

    dx = pxr[...] - pxc[...]
    dy = pyr[...] - pyc[...]
    dd = (dx * dx + dy * dy) + 1e-12
    dist = jnp.sqrt(dd)
    pen = (rc[...] + rr[...]) - dist
    mask = (pen > 0.0) & (jj < ii)

    mi = mask.astype(jnp.int32)
    cnt = jnp.sum(mi, axis=1, keepdims=True)
    k = mi
    for s in (1, 2, 4, 8, 16, 32, 64, 128, 256, 512):
        k = k + _shift_right(k, s, 0)

    cnt_f = jnp.maximum(cnt, 1).astype(jnp.float32)
    q = 1.0 / cnt_f
    c = k.astype(jnp.float32) * q
    r = (cnt_f * q) * omu
    ge = c >= r
    prev = _shift_right(c, 1, jnp.float32(0))
    onehot = ge & (prev < r)
    ohf = onehot.astype(jnp.float32)

    jsel = jnp.sum(onehot.astype(jnp.int32) * jj, axis=1, keepdims=True)
    pvx = jnp.sum(ohf * (dx / dist * pen), axis=1, keepdims=True)
    pvy = jnp.sum(ohf * (dy / dist * pen), axis=1, keepdims=True)
    imr = 1.0 / mr[...]
    imj = jnp.sum(ohf * imr, axis=1, keepdims=True)
    imi = 1.0 / mc[...]
    valid = cnt > 0

    nden = jnp.sqrt(pvx * pvx + pvy * pvy) + 1e-12
    nxv = pvx / nden
    nyv = pvy / nden
    s = imi + imj
    avi = 1.5 * imi / s
    avj = 1.5 * imj / s
    corrx = (0.8 * pvx) / s
    corry = (0.8 * pvy) / s

    icol = i0 + lax.broadcasted_iota(jnp.int32, (_B, 1), 0)
    js_ref[...] = jnp.where(valid, jsel, icol)
    nx_ref[...] = nxv
    ny_ref[...] = nyv
    anx_ref[...] = avi * nxv
    any_ref[...] = avi * nyv
    bnx_ref[...] = avj * nxv
    bny_ref[...] = avj * nyv
    dpix_ref[...] = -(corrx * imi)
    dpiy_ref[...] = -(corry * imi)
    dpjx_ref[...] = corrx * imj
    dpjy_ref[...] = corry * imj


def _detect(omu, pxr, pyr, rr, mr, pxc, pyc, rc, mc):
    row = pl.BlockSpec((1, _N), lambda g: (0, 0))
    col = pl.BlockSpec((_B, 1), lambda g: (g, 0))
    out1 = pl.BlockSpec((_B, 1), lambda g: (g, 0))
    f1 = jax.ShapeDtypeStruct((_N, 1), jnp.float32)
    return pl.pallas_call(
        _detect_body,
        grid=(_N // _B,),
        in_specs=[pl.BlockSpec(memory_space=pltpu.SMEM),
                  row, row, row, row, col, col, col, col],
        out_specs=[out1] * 11,
        out_shape=[jax.ShapeDtypeStruct((_N, 1), jnp.int32)] + [f1] * 10,
    )(omu, pxr, pyr, rr, mr, pxc, pyc, rc, mc)


def _resolve_body(s_hbm, js_hbm, w_hbm, a_hbm, d_hbm, out_hbm,
                  s_v, js_v, w_v, a_v, d_v):
    cid = lax.axis_index("c")
    sid = lax.axis_index("s")

    @pl.when((cid == 0) & (sid == 0))
    def _():
        pltpu.sync_copy(s_hbm, s_v)
        pltpu.sync_copy(js_hbm, js_v)
        pltpu.sync_copy(w_hbm, w_v)
        pltpu.sync_copy(a_hbm, a_v)
        pltpu.sync_copy(d_hbm, d_v)

        lane = lax.iota(jnp.int32, _L)
        maski = jnp.where((lane < 4) | (lane >= 8), 1, 0)
        maskj = jnp.where((lane >= 4) & (lane < 8), 1, 0)
        offs = jnp.where(lane < 8, (lane & 3) * _N, 0)
        mask8 = lane < 8
        zero = jnp.zeros((_L,), jnp.float32)

        def body(i, carry):
            ivec = jnp.full((_L,), i, jnp.int32)
            jvec = plsc.load_gather(js_v, [ivec])
            idx = ivec * maski + jvec * maskj + offs
            state = plsc.load_gather(s_v, [idx])
            base = i * _L
            w = w_v[pl.ds(base, _L)]
            avn = a_v[pl.ds(base, _L)]
            dp = d_v[pl.ds(base, _L)]
            vn = jnp.sum(w * state)
            vnb = jnp.full((_L,), vn)
            dv = jnp.where(vnb < 0.0, vnb * avn, zero)
            plsc.store_scatter(s_v, [idx], state + dp + dv, mask=mask8)
            return carry

        lax.fori_loop(0, _N, body, 0)
        pltpu.sync_copy(s_v, out_hbm)


def _resolve(*args):
    fn = functools.partial(
        pl.kernel,
        out_type=jax.ShapeDtypeStruct((4 * _N,), jnp.float32),
        mesh=plsc.VectorSubcoreMesh(core_axis_name="c", subcore_axis_name="s"),
        scratch_types=[
            pltpu.VMEM((4 * _N,), jnp.float32),
            pltpu.VMEM((_N,), jnp.int32),
            pltpu.VMEM((_N * _L,), jnp.float32),
            pltpu.VMEM((_N * _L,), jnp.float32),
            pltpu.VMEM((_N * _L,), jnp.float32),
        ],
        compiler_params=pltpu.CompilerParams(needs_layout_passes=False),
    )(_resolve_body)
    return fn(*args)


def kernel(positions, velocities, radii, masses):
    px = positions[:, 0]
    py = positions[:, 1]
    omu = (1.0 - jax.random.uniform(jax.random.PRNGKey(0), (), jnp.float32))
    omu = omu.reshape(1, 1)

    outs = _detect(
        omu,
        px.reshape(1, _N), py.reshape(1, _N),
        radii.reshape(1, _N), masses.reshape(1, _N),
        px.reshape(_N, 1), py.reshape(_N, 1),
        radii.reshape(_N, 1), masses.reshape(_N, 1),
    )
    js, nx, ny, anx, any_, bnx, bny, dpix, dpiy, dpjx, dpjy = outs

    z = jnp.zeros((_N, 1), jnp.float32)
    w_rows = jnp.concatenate(
        [z, z, -nx, -ny, z, z, nx, ny] + [z] * 8, axis=1)
    a_rows = jnp.concatenate(
        [z, z, anx, any_, z, z, -bnx, -bny] + [z] * 8, axis=1)
    d_rows = jnp.concatenate(
        [dpix, dpiy, z, z, dpjx, dpjy, z, z] + [z] * 8, axis=1)

    s0 = jnp.concatenate([px, py, velocities[:, 0], velocities[:, 1]])
    s_out = _resolve(s0, js.reshape(_N), w_rows.reshape(_N * _L),
                     a_rows.reshape(_N * _L), d_rows.reshape(_N * _L))
    return jnp.stack(
        [s_out[0:_N], s_out[_N:2 * _N], s_out[2 * _N:3 * _N],
         s_out[3 * _N:4 * _N]], axis=-1)


# speedup vs baseline: 574.0958x; 1.3726x over previous
import functools

import jax
import jax.numpy as jnp
import numpy as np
from jax import lax
from jax.experimental import pallas as pl
from jax.experimental.pallas import tpu as pltpu
from jax.experimental.pallas import tpu_sc as plsc

_N = 1024
_B = 256
_L = 16

_OMU = float(np.float32(1.0) - np.array(1064475214, np.uint32).view(np.float32))


def _shift_right(x, s, zero):
    pad = jnp.full((x.shape[0], s), zero, x.dtype)
    return jnp.concatenate([pad, x[:, :-s]], axis=1)


def _detect_body(posT, pos, rr, mr, rc, mc,
                 js_ref, valid_ref, w_ref, a_ref, d_ref):
    i0 = pl.program_id(0) * _B
    jj = lax.broadcasted_iota(jnp.int32, (_B, _N), 1)
    ii = i0 + lax.broadcasted_iota(jnp.int32, (_B, _N), 0)

    px_row = posT[0:1, :]
    py_row = posT[1:2, :]
    px_col = pos[:, 0:1]
    py_col = pos[:, 1:2]

    dx = px_row - px_col
    dy = py_row - py_col
    dd = (dx * dx + dy * dy) + 1e-12
    dist = jnp.sqrt(dd)
    pen = (rc[...] + rr[...]) - dist
    mask = (pen > 0.0) & (jj < ii)

    mi = mask.astype(jnp.int32)
    cnt = jnp.sum(mi, axis=1, keepdims=True)
    k = mi
    for s in (1, 2, 4, 8, 16, 32, 64, 128, 256, 512):
        k = k + _shift_right(k, s, 0)

    cnt_f = jnp.maximum(cnt, 1).astype(jnp.float32)
    q = 1.0 / cnt_f
    c = k.astype(jnp.float32) * q
    r = (cnt_f * q) * _OMU
    ge = c >= r
    prev = _shift_right(c, 1, jnp.float32(0))
    onehot = ge & (prev < r)
    ohf = onehot.astype(jnp.float32)

    jsel = jnp.sum(onehot.astype(jnp.int32) * jj, axis=1, keepdims=True)
    pvx = jnp.sum(ohf * (dx / dist * pen), axis=1, keepdims=True)
    pvy = jnp.sum(ohf * (dy / dist * pen), axis=1, keepdims=True)
    imr = 1.0 / mr[...]
    imj = jnp.sum(ohf * imr, axis=1, keepdims=True)
    imi = 1.0 / mc[...]

    nden = jnp.sqrt(pvx * pvx + pvy * pvy) + 1e-12
    nxv = pvx / nden
    nyv = pvy / nden
    s = imi + imj
    avi = 1.5 * imi / s
    avj = 1.5 * imj / s
    corrx = (0.8 * pvx) / s
    corry = (0.8 * pvy) / s

    js_ref[...] = jsel
    valid_ref[...] = (cnt > 0).astype(jnp.int32)
    z = jnp.zeros((_B, 1), jnp.float32)
    z8 = jnp.zeros((_B, 8), jnp.float32)
    w_ref[...] = jnp.concatenate(
        [z, z, -nxv, -nyv, z, z, nxv, nyv, z8], axis=1)
    a_ref[...] = jnp.concatenate(
        [z, z, avi * nxv, avi * nyv, z, z, -(avj * nxv), -(avj * nyv), z8],
        axis=1)
    d_ref[...] = jnp.concatenate(
        [-(corrx * imi), -(corry * imi), z, z, corrx * imj, corry * imj,
         z, z, z8], axis=1)


def _detect(posT, pos, radii, masses):
    row2 = pl.BlockSpec((2, _N), lambda g: (0, 0))
    row1 = pl.BlockSpec((1, _N), lambda g: (0, 0))
    col2 = pl.BlockSpec((_B, 2), lambda g: (g, 0))
    col1 = pl.BlockSpec((_B, 1), lambda g: (g, 0))
    out1 = pl.BlockSpec((_B, 1), lambda g: (g, 0))
    out16 = pl.BlockSpec((_B, _L), lambda g: (g, 0))
    i1 = jax.ShapeDtypeStruct((_N, 1), jnp.int32)
    f16 = jax.ShapeDtypeStruct((_N, _L), jnp.float32)
    return pl.pallas_call(
        _detect_body,
        grid=(_N // _B,),
        in_specs=[row2, col2, row1, row1, col1, col1],
        out_specs=[out1, out1, out16, out16, out16],
        out_shape=[i1, i1, f16, f16, f16],
    )(posT, pos, radii.reshape(1, _N), masses.reshape(1, _N),
      radii.reshape(_N, 1), masses.reshape(_N, 1))


def _resolve_body(pos_hbm, vel_hbm, js_hbm, valid_hbm, w_hbm, a_hbm, d_hbm,
                  pos_out, vel_out,
                  s_v, js_v, valid_v, w_v, a_v, d_v, crow_v, cjs_v):
    cid = lax.axis_index("c")
    sid = lax.axis_index("s")

    @pl.when((cid == 0) & (sid == 0))
    def _():
        pltpu.sync_copy(pos_hbm, s_v.at[pl.ds(0, 2 * _N)])
        pltpu.sync_copy(vel_hbm, s_v.at[pl.ds(2 * _N, 2 * _N)])
        pltpu.sync_copy(js_hbm, js_v)
        pltpu.sync_copy(valid_hbm, valid_v)
        pltpu.sync_copy(w_hbm, w_v)
        pltpu.sync_copy(a_hbm, a_v)
        pltpu.sync_copy(d_hbm, d_v)

        lane = lax.iota(jnp.int32, _L)
        maski = jnp.where((lane < 4) | (lane >= 8), 1, 0)
        maskj = jnp.where((lane >= 4) & (lane < 8), 1, 0)
        off2 = jnp.where(
            lane < 8, (lane & 1) + (2 * _N) * ((lane >> 1) & 1), 0)
        mask8 = lane < 8
        zero = jnp.zeros((_L,), jnp.float32)

        def chunk(c, off):
            base = c * _L
            v = valid_v[pl.ds(base, _L)]
            m = v > 0
            rank = (plsc.cumsum(v) - 1) + off
            rows = base + lane
            plsc.store_scatter(crow_v, [rank], rows, mask=m)
            plsc.store_scatter(cjs_v, [rank], js_v[pl.ds(base, _L)], mask=m)
            return off + jnp.sum(v)

        nc = lax.fori_loop(0, _N // _L, chunk, 0)

        def body(t, carry):
            tvec = jnp.full((_L,), t, jnp.int32)
            ivec = plsc.load_gather(crow_v, [tvec])
            jvec = plsc.load_gather(cjs_v, [tvec])
            pidx = ivec * _L + lane
            w = plsc.load_gather(w_v, [pidx])
            avn = plsc.load_gather(a_v, [pidx])
            dp = plsc.load_gather(d_v, [pidx])
            idx = 2 * (ivec * maski + jvec * maskj) + off2
            state = plsc.load_gather(s_v, [idx])
            vn = jnp.sum(w * state)
            vnb = jnp.full((_L,), vn)
            dv = jnp.where(vnb < 0.0, vnb * avn, zero)
            plsc.store_scatter(s_v, [idx], state + dp + dv, mask=mask8)
            return carry

        lax.fori_loop(0, nc, body, 0)

        pltpu.sync_copy(s_v.at[pl.ds(0, 2 * _N)], pos_out)
        pltpu.sync_copy(s_v.at[pl.ds(2 * _N, 2 * _N)], vel_out)


def _resolve(*args):
    fn = functools.partial(
        pl.kernel,
        out_type=[jax.ShapeDtypeStruct((2 * _N,), jnp.float32),
                  jax.ShapeDtypeStruct((2 * _N,), jnp.float32)],
        mesh=plsc.VectorSubcoreMesh(core_axis_name="c", subcore_axis_name="s"),
        scratch_types=[
            pltpu.VMEM((4 * _N,), jnp.float32),
            pltpu.VMEM((_N,), jnp.int32),
            pltpu.VMEM((_N,), jnp.int32),
            pltpu.VMEM((_N * _L,), jnp.float32),
            pltpu.VMEM((_N * _L,), jnp.float32),
            pltpu.VMEM((_N * _L,), jnp.float32),
            pltpu.VMEM((_N,), jnp.int32),
            pltpu.VMEM((_N,), jnp.int32),
        ],
        compiler_params=pltpu.CompilerParams(needs_layout_passes=False),
    )(_resolve_body)
    return fn(*args)


def kernel(positions, velocities, radii, masses):
    posT = positions.T
    js, valid, w_rows, a_rows, d_rows = _detect(posT, positions, radii, masses)
    pos_o, vel_o = _resolve(
        positions.reshape(2 * _N), velocities.reshape(2 * _N),
        js.reshape(_N), valid.reshape(_N),
        w_rows.reshape(_N * _L), a_rows.reshape(_N * _L),
        d_rows.reshape(_N * _L))
    return jnp.concatenate(
        [pos_o.reshape(_N, 2), vel_o.reshape(_N, 2)], axis=-1)


# speedup vs baseline: 688.0008x; 1.1984x over previous
import functools

import jax
import jax.numpy as jnp
import numpy as np
from jax import lax
from jax.experimental import pallas as pl
from jax.experimental.pallas import tpu as pltpu
from jax.experimental.pallas import tpu_sc as plsc

_N = 1024
_B = 512
_L = 16

_OMU = float(np.float32(1.0) - np.array(1064475214, np.uint32).view(np.float32))


def _detect_body(posT, pos, rr, mr, rc, mc, js_ref, valid_ref, rec_ref,
                 lt_ref):
    g = pl.program_id(0)
    i0 = g * _B
    jj = lax.broadcasted_iota(jnp.int32, (_B, _N), 1)
    ii = i0 + lax.broadcasted_iota(jnp.int32, (_B, _N), 0)

    @pl.when(g == 0)
    def _():
        rows = lax.broadcasted_iota(jnp.int32, (_N, _N), 0)
        cols = lax.broadcasted_iota(jnp.int32, (_N, _N), 1)
        lt_ref[...] = (rows <= cols).astype(jnp.bfloat16)

    px_row = posT[0:1, :]
    py_row = posT[1:2, :]
    px_col = pos[:, 0:1]
    py_col = pos[:, 1:2]

    dx = px_row - px_col
    dy = py_row - py_col
    dd = (dx * dx + dy * dy) + 1e-12
    dist = jnp.sqrt(dd)
    pen = (rc[...] + rr[...]) - dist
    mask = (pen > 0.0) & (jj < ii)

    mif = mask.astype(jnp.float32)
    k = lax.dot_general(
        mask.astype(jnp.bfloat16), lt_ref[...],
        (((1,), (0,)), ((), ())), preferred_element_type=jnp.float32)

    cnt = k[:, _N - 1:_N]
    cnt_f = jnp.maximum(cnt, 1.0)
    q = 1.0 / cnt_f
    c = k * q
    r = (cnt_f * q) * _OMU
    onehot = (c >= r) & (((k - mif) * q) < r)
    ohf = onehot.astype(jnp.float32)

    jsel = jnp.sum(onehot.astype(jnp.int32) * jj, axis=1, keepdims=True)
    pvx = jnp.sum(ohf * (dx / dist * pen), axis=1, keepdims=True)
    pvy = jnp.sum(ohf * (dy / dist * pen), axis=1, keepdims=True)
    imr = 1.0 / mr[...]
    imj = jnp.sum(ohf * imr, axis=1, keepdims=True)
    imi = 1.0 / mc[...]

    nden = jnp.sqrt(pvx * pvx + pvy * pvy) + 1e-12
    nxv = pvx / nden
    nyv = pvy / nden
    s = imi + imj
    avi = 1.5 * imi / s
    avj = 1.5 * imj / s
    corrx = (0.8 * pvx) / s
    corry = (0.8 * pvy) / s

    js_ref[...] = jsel
    valid_ref[...] = (cnt > 0.0).astype(jnp.int32)
    z = jnp.zeros((_B, 1), jnp.float32)
    rec_ref[...] = jnp.concatenate(
        [-(corrx * imi), -(corry * imi), avi * nxv, avi * nyv,
         corrx * imj, corry * imj, -(avj * nxv), -(avj * nyv),
         -nxv, -nyv, nxv, nyv, z, z, z, z], axis=1)


def _detect(posT, pos, radii, masses):
    row2 = pl.BlockSpec((2, _N), lambda g: (0, 0))
    row1 = pl.BlockSpec((1, _N), lambda g: (0, 0))
    col2 = pl.BlockSpec((_B, 2), lambda g: (g, 0))
    col1 = pl.BlockSpec((_B, 1), lambda g: (g, 0))
    out1 = pl.BlockSpec((_B, 1), lambda g: (g, 0))
    out16 = pl.BlockSpec((_B, _L), lambda g: (g, 0))
    i1 = jax.ShapeDtypeStruct((_N, 1), jnp.int32)
    f16 = jax.ShapeDtypeStruct((_N, _L), jnp.float32)
    return pl.pallas_call(
        _detect_body,
        grid=(_N // _B,),
        in_specs=[row2, col2, row1, row1, col1, col1],
        out_specs=[out1, out1, out16],
        out_shape=[i1, i1, f16],
        scratch_shapes=[pltpu.VMEM((_N, _N), jnp.bfloat16)],
    )(posT, pos, radii.reshape(1, _N), masses.reshape(1, _N),
      radii.reshape(_N, 1), masses.reshape(_N, 1))


def _resolve_body(pos_hbm, vel_hbm, js_hbm, valid_hbm, rec_hbm,
                  pos_out, vel_out,
                  s_v, js_v, valid_v, rec_v, civ_v):
    cid = lax.axis_index("c")
    sid = lax.axis_index("s")

    @pl.when((cid == 0) & (sid == 0))
    def _():
        pltpu.sync_copy(pos_hbm, s_v.at[pl.ds(0, 2 * _N)])
        pltpu.sync_copy(vel_hbm, s_v.at[pl.ds(2 * _N, 2 * _N)])
        pltpu.sync_copy(js_hbm, js_v)
        pltpu.sync_copy(valid_hbm, valid_v)
        pltpu.sync_copy(rec_hbm, rec_v)

        lane = lax.iota(jnp.int32, _L)
        selpat = (((lane >= 4) & (lane < 8)) | (lane == 10)
                  | (lane == 11)).astype(jnp.int32)
        off3 = jnp.where(
            lane < 8, (lane & 1) + (2 * _N) * ((lane >> 1) & 1),
            jnp.where(lane < 12, (2 * _N) + (lane & 1), 0))
        mask8 = lane < 8
        hi8 = lane >= 8
        vel4 = ((lane & 2) != 0) & mask8
        zero = jnp.zeros((_L,), jnp.float32)

        def chunk(cc, off):
            base = cc * _L
            v = valid_v[pl.ds(base, _L)]
            m = v > 0
            rank2 = ((plsc.cumsum(v) - 1) + off) * 2
            plsc.store_scatter(civ_v, [rank2], base + lane, mask=m)
            plsc.store_scatter(civ_v, [rank2 + 1], js_v[pl.ds(base, _L)],
                               mask=m)
            return off + jnp.sum(v)

        nc = lax.fori_loop(0, _N // _L, chunk, 0)

        def body(t, carry):
            t2 = jnp.full((_L,), 2 * t, jnp.int32)
            ivec = plsc.load_gather(civ_v, [t2])
            sel = plsc.load_gather(civ_v, [t2 + selpat])
            rec = plsc.load_gather(rec_v, [ivec * _L + lane])
            idx = 2 * sel + off3
            state = plsc.load_gather(s_v, [idx])
            tt = rec * state
            vn = jnp.sum(jnp.where(hi8, tt, zero))
            vnb = jnp.full((_L,), vn)
            delta = jnp.where(vel4, jnp.where(vnb < 0.0, vnb * rec, zero),
                              rec)
            plsc.addupdate_scatter(s_v, [idx], delta, mask=mask8)
            return carry

        lax.fori_loop(0, nc, body, 0)

        pltpu.sync_copy(s_v.at[pl.ds(0, 2 * _N)], pos_out)
        pltpu.sync_copy(s_v.at[pl.ds(2 * _N, 2 * _N)], vel_out)


def _resolve(*args):
    fn = functools.partial(
        pl.kernel,
        out_type=[jax.ShapeDtypeStruct((2 * _N,), jnp.float32),
                  jax.ShapeDtypeStruct((2 * _N,), jnp.float32)],
        mesh=plsc.VectorSubcoreMesh(core_axis_name="c", subcore_axis_name="s"),
        scratch_types=[
            pltpu.VMEM((4 * _N,), jnp.float32),
            pltpu.VMEM((_N,), jnp.int32),
            pltpu.VMEM((_N,), jnp.int32),
            pltpu.VMEM((_N * _L,), jnp.float32),
            pltpu.VMEM((2 * _N,), jnp.int32),
        ],
        compiler_params=pltpu.CompilerParams(needs_layout_passes=False),
    )(_resolve_body)
    return fn(*args)


def kernel(positions, velocities, radii, masses):
    posT = positions.T
    js, valid, rec = _detect(posT, positions, radii, masses)
    pos_o, vel_o = _resolve(
        positions.reshape(2 * _N), velocities.reshape(2 * _N),
        js.reshape(_N), valid.reshape(_N), rec.reshape(_N * _L))
    return jnp.concatenate(
        [pos_o.reshape(_N, 2), vel_o.reshape(_N, 2)], axis=-1)


# speedup vs baseline: 722.9271x; 1.0508x over previous
import functools

import jax
import jax.numpy as jnp
import numpy as np
from jax import lax
from jax.experimental import pallas as pl
from jax.experimental.pallas import tpu as pltpu
from jax.experimental.pallas import tpu_sc as plsc

_N = 1024
_B = 512
_L = 16

_OMU = float(np.float32(1.0) - np.array(1064475214, np.uint32).view(np.float32))


def _detect_body(posT, pos, rr, mr, rc, mc, rec_ref, lt_ref):
    g = pl.program_id(0)
    i0 = g * _B
    jj = lax.broadcasted_iota(jnp.int32, (_B, _N), 1)
    ii = i0 + lax.broadcasted_iota(jnp.int32, (_B, _N), 0)

    @pl.when(g == 0)
    def _():
        rows = lax.broadcasted_iota(jnp.int32, (_N, _N), 0)
        cols = lax.broadcasted_iota(jnp.int32, (_N, _N), 1)
        lt_ref[...] = (rows <= cols).astype(jnp.bfloat16)

    px_row = posT[0:1, :]
    py_row = posT[1:2, :]
    px_col = pos[:, 0:1]
    py_col = pos[:, 1:2]

    dx = px_row - px_col
    dy = py_row - py_col
    dd = (dx * dx + dy * dy) + 1e-12
    dist = jnp.sqrt(dd)
    pen = (rc[...] + rr[...]) - dist
    mask = (pen > 0.0) & (jj < ii)

    mif = mask.astype(jnp.float32)
    k = lax.dot_general(
        mask.astype(jnp.bfloat16), lt_ref[...],
        (((1,), (0,)), ((), ())), preferred_element_type=jnp.float32)

    cnt = k[:, _N - 1:_N]
    cnt_f = jnp.maximum(cnt, 1.0)
    q = 1.0 / cnt_f
    c = k * q
    r = (cnt_f * q) * _OMU
    onehot = (c >= r) & (((k - mif) * q) < r)
    ohf = onehot.astype(jnp.float32)

    jsel = jnp.sum(onehot.astype(jnp.int32) * jj, axis=1, keepdims=True)
    dxs = jnp.sum(ohf * dx, axis=1, keepdims=True)
    dys = jnp.sum(ohf * dy, axis=1, keepdims=True)
    dists = jnp.sum(ohf * dist, axis=1, keepdims=True)
    pens = jnp.sum(ohf * pen, axis=1, keepdims=True)
    imr = 1.0 / mr[...]
    imj = jnp.sum(ohf * imr, axis=1, keepdims=True)
    imi = 1.0 / mc[...]

    dists = jnp.where(dists > 0.0, dists, 1.0)
    pvx = dxs / dists * pens
    pvy = dys / dists * pens
    nden = jnp.sqrt(pvx * pvx + pvy * pvy) + 1e-12
    nxv = pvx / nden
    nyv = pvy / nden
    s = imi + imj
    avi = 1.5 * imi / s
    avj = 1.5 * imj / s
    corrx = (0.8 * pvx) / s
    corry = (0.8 * pvy) / s

    z = jnp.zeros((_B, 1), jnp.float32)
    rec_ref[...] = jnp.concatenate(
        [-(corrx * imi), -(corry * imi), avi * nxv, avi * nyv,
         corrx * imj, corry * imj, -(avj * nxv), -(avj * nyv),
         -nxv, -nyv, nxv, nyv,
         jsel.astype(jnp.float32), (cnt > 0.0).astype(jnp.float32),
         z, z], axis=1)


def _detect(posT, pos, radii, masses):
    row2 = pl.BlockSpec((2, _N), lambda g: (0, 0))
    row1 = pl.BlockSpec((1, _N), lambda g: (0, 0))
    col2 = pl.BlockSpec((_B, 2), lambda g: (g, 0))
    col1 = pl.BlockSpec((_B, 1), lambda g: (g, 0))
    out16 = pl.BlockSpec((_B, _L), lambda g: (g, 0))
    f16 = jax.ShapeDtypeStruct((_N, _L), jnp.float32)
    return pl.pallas_call(
        _detect_body,
        grid=(_N // _B,),
        in_specs=[row2, col2, row1, row1, col1, col1],
        out_specs=[out16],
        out_shape=[f16],
        scratch_shapes=[pltpu.VMEM((_N, _N), jnp.bfloat16)],
    )(posT, pos, radii.reshape(1, _N), masses.reshape(1, _N),
      radii.reshape(_N, 1), masses.reshape(_N, 1))[0]


def _resolve_body(pos_hbm, vel_hbm, rec_hbm,
                  pos_out, vel_out,
                  s_v, rec_v, civ_v):
    cid = lax.axis_index("c")
    sid = lax.axis_index("s")

    @pl.when((cid == 0) & (sid == 0))
    def _():
        pltpu.sync_copy(pos_hbm, s_v.at[pl.ds(0, 2 * _N)])
        pltpu.sync_copy(vel_hbm, s_v.at[pl.ds(2 * _N, 2 * _N)])
        pltpu.sync_copy(rec_hbm, rec_v)

        lane = lax.iota(jnp.int32, _L)
        selpat = (((lane >= 4) & (lane < 8)) | (lane == 10)
                  | (lane == 11)).astype(jnp.int32)
        off3 = jnp.where(
            lane < 8, (lane & 1) + (2 * _N) * ((lane >> 1) & 1),
            jnp.where(lane < 12, (2 * _N) + (lane & 1), 0))
        mask8 = lane < 8
        hi8 = (lane >= 8) & (lane < 12)
        vel4 = ((lane & 2) != 0) & mask8
        zero = jnp.zeros((_L,), jnp.float32)

        def chunk(cc, off):
            rows = cc * _L + lane
            vf = plsc.load_gather(rec_v, [rows * _L + 13])
            jf = plsc.load_gather(rec_v, [rows * _L + 12])
            v = vf.astype(jnp.int32)
            m = v > 0
            rank2 = ((plsc.cumsum(v) - 1) + off) * 2
            plsc.store_scatter(civ_v, [rank2], rows, mask=m)
            plsc.store_scatter(civ_v, [rank2 + 1], jf.astype(jnp.int32),
                               mask=m)
            return off + jnp.sum(v)

        nc = lax.fori_loop(0, _N // _L, chunk, 0)

        def body(t, carry):
            t2 = jnp.full((_L,), 2 * t, jnp.int32)
            ivec = plsc.load_gather(civ_v, [t2])
            sel = plsc.load_gather(civ_v, [t2 + selpat])
            rec = plsc.load_gather(rec_v, [ivec * _L + lane])
            idx = 2 * sel + off3
            state = plsc.load_gather(s_v, [idx])
            tt = rec * state
            vn = jnp.sum(jnp.where(hi8, tt, zero))
            vnb = jnp.full((_L,), vn)
            delta = jnp.where(vel4, jnp.where(vnb < 0.0, vnb * rec, zero),
                              rec)
            plsc.addupdate_scatter(s_v, [idx], delta, mask=mask8)
            return carry

        lax.fori_loop(0, nc, body, 0)

        pltpu.sync_copy(s_v.at[pl.ds(0, 2 * _N)], pos_out)
        pltpu.sync_copy(s_v.at[pl.ds(2 * _N, 2 * _N)], vel_out)


def _resolve(*args):
    fn = functools.partial(
        pl.kernel,
        out_type=[jax.ShapeDtypeStruct((2 * _N,), jnp.float32),
                  jax.ShapeDtypeStruct((2 * _N,), jnp.float32)],
        mesh=plsc.VectorSubcoreMesh(core_axis_name="c", subcore_axis_name="s"),
        scratch_types=[
            pltpu.VMEM((4 * _N,), jnp.float32),
            pltpu.VMEM((_N * _L,), jnp.float32),
            pltpu.VMEM((2 * _N,), jnp.int32),
        ],
        compiler_params=pltpu.CompilerParams(needs_layout_passes=False),
    )(_resolve_body)
    return fn(*args)


def kernel(positions, velocities, radii, masses):
    posT = positions.T
    rec = _detect(posT, positions, radii, masses)
    pos_o, vel_o = _resolve(
        positions.reshape(2 * _N), velocities.reshape(2 * _N),
        rec.reshape(_N * _L))
    return jnp.concatenate(
        [pos_o.reshape(_N, 2), vel_o.reshape(_N, 2)], axis=-1)
